# baseline (speedup 1.0000x reference)
import jax
import jax.numpy as jnp
from jax import lax
from jax.experimental import pallas as pl
from jax.experimental.pallas import tpu as pltpu
from jax.experimental.pallas import tpu_sc as plsc

N = 4194304

NC = 2
NS = 16
NW = NC * NS
PW = N // NW
CHUNK = 8192
NCHUNK = PW // CHUNK
UNROLL = 8

NBITS = 15
B = 1 << NBITS
HR = 256
SHIFT = 32 - NBITS - 1

MIN_NEG = 41943

_LOG1P = (3.5110213048028527e-06, 0.9997923374176025, -0.49697741866111755,
          0.31458917260169983, -0.18878082931041718, 0.08172564208507538,
          -0.01720779947936535)


def _sc_main_body(x_hbm, t_hbm, ocnt_hbm, osum_hbm,
                  bx0, bx1, bt0, bt1, hcnt, hsum,
                  sx0, sx1, st0, st1):
    c = lax.axis_index("c")
    s = lax.axis_index("s")
    wid = s * NC + c
    base = wid * PW

    zeros16 = jnp.zeros((16,), jnp.float32)
    ones16 = jnp.ones((16,), jnp.float32)

    @plsc.parallel_loop(0, HR, unroll=4)
    def zbody(j):
        for u in range(8):
            hcnt[j, pl.ds(u * 16, 16)] = zeros16
            hsum[j, pl.ds(u * 16, 16)] = zeros16

    bxs = (bx0, bx1)
    bts = (bt0, bt1)
    sxs = (sx0, sx1)
    sts = (st0, st1)

    def start(ci):
        sl = pl.ds(base + ci * CHUNK, CHUNK)
        return (pltpu.async_copy(x_hbm.at[sl], bxs[ci % 2], sxs[ci % 2]),
                pltpu.async_copy(t_hbm.at[sl], bts[ci % 2], sts[ci % 2]))

    def process(bufx, buft, carry):
        @plsc.parallel_loop(0, CHUNK // 16, unroll=UNROLL, carry=carry)
        def ibody(i, cr):
            ps, npv = cr
            x = bufx[pl.ds(i * 16, 16)]
            t = buft[pl.ds(i * 16, 16)]
            e = jnp.exp(-jnp.abs(x))
            p = jnp.float32(_LOG1P[6])
            for cf in _LOG1P[5::-1]:
                p = p * e + jnp.float32(cf)
            loss = jnp.maximum(jnp.maximum(x, 0.0) - x * t + p, 0.0)
            m = t == 0.0
            bits = plsc.bitcast(loss, jnp.int32)
            idx = lax.shift_right_logical(bits, SHIFT)
            hi = lax.shift_right_logical(idx, 7)
            lo = idx & 127
            plsc.addupdate_scatter(hcnt, [hi, lo], ones16, mask=m)
            plsc.addupdate_scatter(hsum, [hi, lo], loss, mask=m)
            return (ps + loss * t, npv + t)

        return ibody

    carry = (zeros16, zeros16)
    copies = [None, None]
    copies[0] = start(0)
    for ci in range(NCHUNK):
        if ci + 1 < NCHUNK:
            copies[(ci + 1) % 2] = start(ci + 1)
        cx, ct = copies[ci % 2]
        cx.wait()
        ct.wait()
        carry = process(bxs[ci % 2], bts[ci % 2], carry)
    ps16, np16 = carry

    hcnt[HR - 1, pl.ds(112, 16)] = np16
    hsum[HR - 1, pl.ds(112, 16)] = ps16

    pltpu.sync_copy(hcnt, ocnt_hbm.at[wid])
    pltpu.sync_copy(hsum, osum_hbm.at[wid])


def _sc_hist(x, t):
    mesh = plsc.VectorSubcoreMesh(core_axis_name="c", subcore_axis_name="s")
    f = pl.kernel(
        _sc_main_body,
        out_type=[
            jax.ShapeDtypeStruct((NW, HR, 128), jnp.float32),
            jax.ShapeDtypeStruct((NW, HR, 128), jnp.float32),
        ],
        mesh=mesh,
        compiler_params=pltpu.CompilerParams(needs_layout_passes=False),
        scratch_types=[
            pltpu.VMEM((CHUNK,), jnp.float32),
            pltpu.VMEM((CHUNK,), jnp.float32),
            pltpu.VMEM((CHUNK,), jnp.float32),
            pltpu.VMEM((CHUNK,), jnp.float32),
            pltpu.VMEM((HR, 128), jnp.float32),
            pltpu.VMEM((HR, 128), jnp.float32),
            pltpu.SemaphoreType.DMA,
            pltpu.SemaphoreType.DMA,
            pltpu.SemaphoreType.DMA,
            pltpu.SemaphoreType.DMA,
        ],
    )
    return f(x, t)


def _k3_select(cnt_ref, sm_ref, out_ref):
    cnt = jnp.sum(cnt_ref[...], axis=0)
    sm = jnp.sum(sm_ref[...], axis=0)

    r0 = lax.broadcasted_iota(jnp.int32, (HR, 128), 0)
    c0 = lax.broadcasted_iota(jnp.int32, (HR, 128), 1)
    stash = (r0 == HR - 1) & (c0 >= 112)
    npos = jnp.sum(jnp.where(stash, cnt, 0.0))
    sum_pos = jnp.sum(jnp.where(stash, sm, 0.0))
    cnt = jnp.where(stash, 0.0, cnt)
    sm = jnp.where(stash, 0.0, sm)

    col = lax.broadcasted_iota(jnp.int32, (128, 128), 0)
    row = lax.broadcasted_iota(jnp.int32, (128, 128), 1)
    upper = (col <= row).astype(jnp.float32)
    incl_row = lax.dot(cnt, upper, precision=lax.Precision.HIGHEST,
                       preferred_element_type=jnp.float32)
    row_tot = incl_row[:, 127:128]
    i2 = lax.broadcasted_iota(jnp.int32, (HR, HR), 0)
    j2 = lax.broadcasted_iota(jnp.int32, (HR, HR), 1)
    lstrict = (j2 < i2).astype(jnp.float32)
    pref_rows = lax.dot(lstrict, row_tot, precision=lax.Precision.HIGHEST,
                        preferred_element_type=jnp.float32)
    incl = incl_row + pref_rows
    tot = jnp.sum(cnt)
    above = tot - incl

    npi = npos.astype(jnp.int32)
    maxneg = N - npi
    k = jnp.minimum(jnp.maximum(MIN_NEG, 5 * npi), maxneg)
    kf = k.astype(jnp.float32)

    w = jnp.clip((kf - above) / cnt, 0.0, 1.0)
    w = jnp.where(cnt > 0.0, w, 0.0)
    sum_neg = jnp.sum(sm * w)
    count = npos + kf
    out_ref[0, 0] = (sum_pos + sum_neg) / count


def kernel(input, target):
    ocnt, osum = _sc_hist(input, target)

    out = pl.pallas_call(
        _k3_select,
        in_specs=[
            pl.BlockSpec((NW, HR, 128), lambda: (0, 0, 0)),
            pl.BlockSpec((NW, HR, 128), lambda: (0, 0, 0)),
        ],
        out_specs=pl.BlockSpec(memory_space=pltpu.SMEM),
        out_shape=jax.ShapeDtypeStruct((1, 1), jnp.float32),
    )(ocnt, osum)

    return out[0, 0]


# speedup vs baseline: 1.0923x; 1.0923x over previous
import jax
import jax.numpy as jnp
from jax import lax
from jax.experimental import pallas as pl
from jax.experimental.pallas import tpu as pltpu
from jax.experimental.pallas import tpu_sc as plsc

N = 4194304
GRID1 = 8
BLK1 = N // GRID1

NC = 2
NS = 16
NW = NC * NS
PW = N // NW
CHUNK = 8192
NCHUNK = PW // CHUNK
UNROLL = 8

NBITS = 15
B = 1 << NBITS
HR = 256
SHIFT = 32 - NBITS - 1

MIN_NEG = 41943


def _k1_bce(x_ref, t_ref, nl_ref, sp_ref, np_ref):
    i = pl.program_id(0)
    x = x_ref[...].reshape(BLK1 // 128, 128)
    t = t_ref[...].reshape(BLK1 // 128, 128)
    loss = jnp.maximum(x, 0.0) - x * t + jnp.log1p(jnp.exp(-jnp.abs(x)))
    nl_ref[...] = jnp.where(t == 0.0, loss, 0.0).reshape(BLK1)
    ps = jnp.sum(loss * t)
    npos = jnp.sum(t)

    @pl.when(i == 0)
    def _():
        sp_ref[0, 0] = ps
        np_ref[0, 0] = npos

    @pl.when(i > 0)
    def _():
        sp_ref[0, 0] += ps
        np_ref[0, 0] += npos


def _sc_hist_body(nl_hbm, ocnt_hbm, osum_hbm, buf0, buf1, hcnt, hsum,
                  shcnt, shsum, rows, sem0, sem1):
    c = lax.axis_index("c")
    s = lax.axis_index("s")
    wid = s * NC + c
    base = wid * PW

    zeros16 = jnp.zeros((16,), jnp.float32)
    ones16 = jnp.ones((16,), jnp.float32)
    iota16 = lax.iota(jnp.int32, 16)

    @plsc.parallel_loop(0, HR, unroll=4)
    def zbody(j):
        for u in range(8):
            hcnt[j, pl.ds(u * 16, 16)] = zeros16
            hsum[j, pl.ds(u * 16, 16)] = zeros16

    for j in range(2):
        for u in range(8):
            rows[j, pl.ds(u * 16, 16)] = iota16 + (j * 128 + u * 16)

    @pl.when(s == 0)
    def _():
        pltpu.sync_copy(hcnt, shcnt)
        pltpu.sync_copy(hsum, shsum)

    bufs = (buf0, buf1)
    sems = (sem0, sem1)

    def start(ci):
        return pltpu.async_copy(
            nl_hbm.at[pl.ds(base + ci * CHUNK, CHUNK)],
            bufs[ci % 2], sems[ci % 2])

    def process(buf):
        @plsc.parallel_loop(0, CHUNK // 16, unroll=UNROLL)
        def ibody(i):
            v = buf[pl.ds(i * 16, 16)]
            m = v > 0.0
            bits = plsc.bitcast(v, jnp.int32)
            idx = lax.shift_right_logical(bits, SHIFT)
            hi = lax.shift_right_logical(idx, 7)
            lo = idx & 127
            plsc.addupdate_scatter(hcnt, [hi, lo], ones16, mask=m)
            plsc.addupdate_scatter(hsum, [hi, lo], v, mask=m)

    copies = [None, None]
    copies[0] = start(0)
    for ci in range(NCHUNK):
        if ci + 1 < NCHUNK:
            copies[(ci + 1) % 2] = start(ci + 1)
        copies[ci % 2].wait()
        process(bufs[ci % 2])

    plsc.subcore_barrier()
    for j in range(2):
        sl = pl.ds(j * 128, 128)
        pltpu.sync_copy(hcnt.at[sl], shcnt.at[rows.at[j]], add=True)
        pltpu.sync_copy(hsum.at[sl], shsum.at[rows.at[j]], add=True)
    plsc.subcore_barrier()

    @pl.when(s == 0)
    def _():
        pltpu.sync_copy(shcnt, ocnt_hbm.at[c])
        pltpu.sync_copy(shsum, osum_hbm.at[c])


def _sc_hist(nl_flat):
    mesh = plsc.VectorSubcoreMesh(core_axis_name="c", subcore_axis_name="s")
    f = pl.kernel(
        _sc_hist_body,
        out_type=[
            jax.ShapeDtypeStruct((NC, HR, 128), jnp.float32),
            jax.ShapeDtypeStruct((NC, HR, 128), jnp.float32),
        ],
        mesh=mesh,
        compiler_params=pltpu.CompilerParams(needs_layout_passes=False),
        scratch_types=[
            pltpu.VMEM((CHUNK,), jnp.float32),
            pltpu.VMEM((CHUNK,), jnp.float32),
            pltpu.VMEM((HR, 128), jnp.float32),
            pltpu.VMEM((HR, 128), jnp.float32),
            pltpu.VMEM_SHARED((HR, 128), jnp.float32),
            pltpu.VMEM_SHARED((HR, 128), jnp.float32),
            pltpu.VMEM((2, 128), jnp.int32),
            pltpu.SemaphoreType.DMA,
            pltpu.SemaphoreType.DMA,
        ],
    )
    return f(nl_flat)


def _k3_select(cnt_ref, sm_ref, sp_ref, np_ref, out_ref):
    npos = np_ref[0, 0]
    cnt = jnp.sum(cnt_ref[...], axis=0)
    sm = jnp.sum(sm_ref[...], axis=0)

    col = lax.broadcasted_iota(jnp.int32, (128, 128), 0)
    row = lax.broadcasted_iota(jnp.int32, (128, 128), 1)
    upper = (col <= row).astype(jnp.float32)
    incl_row = lax.dot(cnt, upper, precision=lax.Precision.HIGHEST,
                       preferred_element_type=jnp.float32)
    row_tot = incl_row[:, 127:128]
    i2 = lax.broadcasted_iota(jnp.int32, (HR, HR), 0)
    j2 = lax.broadcasted_iota(jnp.int32, (HR, HR), 1)
    lstrict = (j2 < i2).astype(jnp.float32)
    pref_rows = lax.dot(lstrict, row_tot, precision=lax.Precision.HIGHEST,
                        preferred_element_type=jnp.float32)
    incl = incl_row + pref_rows
    tot = jnp.sum(cnt)
    above = tot - incl

    npi = npos.astype(jnp.int32)
    maxneg = N - npi
    k = jnp.minimum(jnp.maximum(MIN_NEG, 5 * npi), maxneg)
    kf = k.astype(jnp.float32)

    w = jnp.clip((kf - above) / cnt, 0.0, 1.0)
    w = jnp.where(cnt > 0.0, w, 0.0)
    sum_neg = jnp.sum(sm * w)
    count = npos + kf
    out_ref[0, 0] = (sp_ref[0, 0] + sum_neg) / count


def kernel(input, target):
    nl, sp, npos = pl.pallas_call(
        _k1_bce,
        grid=(GRID1,),
        in_specs=[
            pl.BlockSpec((BLK1,), lambda i: (i,)),
            pl.BlockSpec((BLK1,), lambda i: (i,)),
        ],
        out_specs=[
            pl.BlockSpec((BLK1,), lambda i: (i,)),
            pl.BlockSpec(memory_space=pltpu.SMEM),
            pl.BlockSpec(memory_space=pltpu.SMEM),
        ],
        out_shape=[
            jax.ShapeDtypeStruct((N,), jnp.float32),
            jax.ShapeDtypeStruct((1, 1), jnp.float32),
            jax.ShapeDtypeStruct((1, 1), jnp.float32),
        ],
    )(input, target)

    ocnt, osum = _sc_hist(nl)

    out = pl.pallas_call(
        _k3_select,
        in_specs=[
            pl.BlockSpec((NC, HR, 128), lambda: (0, 0, 0)),
            pl.BlockSpec((NC, HR, 128), lambda: (0, 0, 0)),
            pl.BlockSpec(memory_space=pltpu.SMEM),
            pl.BlockSpec(memory_space=pltpu.SMEM),
        ],
        out_specs=pl.BlockSpec(memory_space=pltpu.SMEM),
        out_shape=jax.ShapeDtypeStruct((1, 1), jnp.float32),
    )(ocnt, osum, sp, npos)

    return out[0, 0]


# speedup vs baseline: 1.0996x; 1.0066x over previous
import jax
import jax.numpy as jnp
from jax import lax
from jax.experimental import pallas as pl
from jax.experimental.pallas import tpu as pltpu
from jax.experimental.pallas import tpu_sc as plsc

N = 4194304
GRID1 = 4
BLK1 = N // GRID1

NC = 2
NS = 16
NW = NC * NS
PW = N // NW
CHUNK = 16384
NCHUNK = PW // CHUNK
UNROLL = 8

NBITS = 15
B = 1 << NBITS
HR = 256
SHIFT = 32 - NBITS - 1

MIN_NEG = 41943


def _k1_bce(x_ref, t_ref, nl_ref, sp_ref, np_ref):
    i = pl.program_id(0)
    x = x_ref[...].reshape(BLK1 // 128, 128)
    t = t_ref[...].reshape(BLK1 // 128, 128)
    loss = jnp.maximum(x, 0.0) - x * t + jnp.log1p(jnp.exp(-jnp.abs(x)))
    nl_ref[...] = jnp.where(t == 0.0, loss, 0.0).reshape(BLK1)
    ps = jnp.sum(loss * t)
    npos = jnp.sum(t)

    @pl.when(i == 0)
    def _():
        sp_ref[0, 0] = ps
        np_ref[0, 0] = npos

    @pl.when(i > 0)
    def _():
        sp_ref[0, 0] += ps
        np_ref[0, 0] += npos


def _sc_hist_body(nl_hbm, ocnt_hbm, osum_hbm, buf0, buf1, hcnt, hsum,
                  shcnt, shsum, rows, sem0, sem1):
    c = lax.axis_index("c")
    s = lax.axis_index("s")
    wid = s * NC + c
    base = wid * PW

    zeros16 = jnp.zeros((16,), jnp.float32)
    ones16 = jnp.ones((16,), jnp.float32)
    iota16 = lax.iota(jnp.int32, 16)

    @plsc.parallel_loop(0, HR, unroll=4)
    def zbody(j):
        for u in range(8):
            hcnt[j, pl.ds(u * 16, 16)] = zeros16
            hsum[j, pl.ds(u * 16, 16)] = zeros16

    for j in range(2):
        for u in range(8):
            rows[j, pl.ds(u * 16, 16)] = iota16 + (j * 128 + u * 16)

    @pl.when(s == 0)
    def _():
        pltpu.sync_copy(hcnt, shcnt)
        pltpu.sync_copy(hsum, shsum)

    bufs = (buf0, buf1)
    sems = (sem0, sem1)

    def start(ci):
        return pltpu.async_copy(
            nl_hbm.at[pl.ds(base + ci * CHUNK, CHUNK)],
            bufs[ci % 2], sems[ci % 2])

    def process(buf):
        @plsc.parallel_loop(0, CHUNK // 16, unroll=UNROLL)
        def ibody(i):
            v = buf[pl.ds(i * 16, 16)]
            m = v > 0.0
            bits = plsc.bitcast(v, jnp.int32)
            idx = lax.shift_right_logical(bits, SHIFT)
            hi = lax.shift_right_logical(idx, 7)
            lo = idx & 127
            plsc.addupdate_scatter(hcnt, [hi, lo], ones16, mask=m)
            plsc.addupdate_scatter(hsum, [hi, lo], v, mask=m)

    copies = [None, None]
    copies[0] = start(0)
    for ci in range(NCHUNK):
        if ci + 1 < NCHUNK:
            copies[(ci + 1) % 2] = start(ci + 1)
        copies[ci % 2].wait()
        process(bufs[ci % 2])

    plsc.subcore_barrier()
    for j in range(2):
        sl = pl.ds(j * 128, 128)
        pltpu.sync_copy(hcnt.at[sl], shcnt.at[rows.at[j]], add=True)
        pltpu.sync_copy(hsum.at[sl], shsum.at[rows.at[j]], add=True)
    plsc.subcore_barrier()

    @pl.when(s == 0)
    def _():
        pltpu.sync_copy(shcnt, ocnt_hbm.at[c])
        pltpu.sync_copy(shsum, osum_hbm.at[c])


def _sc_hist(nl_flat):
    mesh = plsc.VectorSubcoreMesh(core_axis_name="c", subcore_axis_name="s")
    f = pl.kernel(
        _sc_hist_body,
        out_type=[
            jax.ShapeDtypeStruct((NC, HR, 128), jnp.float32),
            jax.ShapeDtypeStruct((NC, HR, 128), jnp.float32),
        ],
        mesh=mesh,
        compiler_params=pltpu.CompilerParams(needs_layout_passes=False),
        scratch_types=[
            pltpu.VMEM((CHUNK,), jnp.float32),
            pltpu.VMEM((CHUNK,), jnp.float32),
            pltpu.VMEM((HR, 128), jnp.float32),
            pltpu.VMEM((HR, 128), jnp.float32),
            pltpu.VMEM_SHARED((HR, 128), jnp.float32),
            pltpu.VMEM_SHARED((HR, 128), jnp.float32),
            pltpu.VMEM((2, 128), jnp.int32),
            pltpu.SemaphoreType.DMA,
            pltpu.SemaphoreType.DMA,
        ],
    )
    return f(nl_flat)


def _k3_select(cnt_ref, sm_ref, sp_ref, np_ref, out_ref):
    npos = np_ref[0, 0]
    cnt = jnp.sum(cnt_ref[...], axis=0)
    sm = jnp.sum(sm_ref[...], axis=0)

    col = lax.broadcasted_iota(jnp.int32, (128, 128), 0)
    row = lax.broadcasted_iota(jnp.int32, (128, 128), 1)
    upper = (col <= row).astype(jnp.float32)
    incl_row = lax.dot(cnt, upper, precision=lax.Precision.HIGHEST,
                       preferred_element_type=jnp.float32)
    row_tot = incl_row[:, 127:128]
    i2 = lax.broadcasted_iota(jnp.int32, (HR, HR), 0)
    j2 = lax.broadcasted_iota(jnp.int32, (HR, HR), 1)
    lstrict = (j2 < i2).astype(jnp.float32)
    pref_rows = lax.dot(lstrict, row_tot, precision=lax.Precision.HIGHEST,
                        preferred_element_type=jnp.float32)
    incl = incl_row + pref_rows
    tot = jnp.sum(cnt)
    above = tot - incl

    npi = npos.astype(jnp.int32)
    maxneg = N - npi
    k = jnp.minimum(jnp.maximum(MIN_NEG, 5 * npi), maxneg)
    kf = k.astype(jnp.float32)

    w = jnp.clip((kf - above) / cnt, 0.0, 1.0)
    w = jnp.where(cnt > 0.0, w, 0.0)
    sum_neg = jnp.sum(sm * w)
    count = npos + kf
    out_ref[0, 0] = (sp_ref[0, 0] + sum_neg) / count


def kernel(input, target):
    nl, sp, npos = pl.pallas_call(
        _k1_bce,
        grid=(GRID1,),
        in_specs=[
            pl.BlockSpec((BLK1,), lambda i: (i,)),
            pl.BlockSpec((BLK1,), lambda i: (i,)),
        ],
        out_specs=[
            pl.BlockSpec((BLK1,), lambda i: (i,)),
            pl.BlockSpec(memory_space=pltpu.SMEM),
            pl.BlockSpec(memory_space=pltpu.SMEM),
        ],
        out_shape=[
            jax.ShapeDtypeStruct((N,), jnp.float32),
            jax.ShapeDtypeStruct((1, 1), jnp.float32),
            jax.ShapeDtypeStruct((1, 1), jnp.float32),
        ],
    )(input, target)

    ocnt, osum = _sc_hist(nl)

    out = pl.pallas_call(
        _k3_select,
        in_specs=[
            pl.BlockSpec((NC, HR, 128), lambda: (0, 0, 0)),
            pl.BlockSpec((NC, HR, 128), lambda: (0, 0, 0)),
            pl.BlockSpec(memory_space=pltpu.SMEM),
            pl.BlockSpec(memory_space=pltpu.SMEM),
        ],
        out_specs=pl.BlockSpec(memory_space=pltpu.SMEM),
        out_shape=jax.ShapeDtypeStruct((1, 1), jnp.float32),
    )(ocnt, osum, sp, npos)

    return out[0, 0]


# speedup vs baseline: 1.1682x; 1.0624x over previous
import jax
import jax.numpy as jnp
from jax import lax
from jax.experimental import pallas as pl
from jax.experimental.pallas import tpu as pltpu
from jax.experimental.pallas import tpu_sc as plsc

N = 4194304
GRID1 = 8
BLK1 = N // GRID1

NC = 2
NS = 16
NW = NC * NS
PW = N // NW
CHUNK = 8192
NCHUNK = PW // CHUNK
UNROLL = 4

NBITS = 15
B = 1 << NBITS
HR = 256
SHIFT = 32 - NBITS - 1

MIN_NEG = 41943


def _k1_bce(x_ref, t_ref, nl_ref, sp_ref, np_ref):
    i = pl.program_id(0)
    x = x_ref[...].reshape(BLK1 // 128, 128)
    t = t_ref[...].reshape(BLK1 // 128, 128)
    loss = jnp.maximum(x, 0.0) - x * t + jnp.log1p(jnp.exp(-jnp.abs(x)))
    nl = jnp.where(t == 0.0, loss, 0.0).astype(jnp.bfloat16)
    nl_ref[...] = nl.reshape(BLK1)
    ps = jnp.sum(loss * t)
    npos = jnp.sum(t)

    @pl.when(i == 0)
    def _():
        sp_ref[0, 0] = ps
        np_ref[0, 0] = npos

    @pl.when(i > 0)
    def _():
        sp_ref[0, 0] += ps
        np_ref[0, 0] += npos


def _sc_hist_body(nl_hbm, ocnt_hbm, osum_hbm, buf0, buf1, hcnt, hsum,
                  sem0, sem1):
    c = lax.axis_index("c")
    s = lax.axis_index("s")
    wid = s * NC + c
    base = wid * PW

    zeros16 = jnp.zeros((16,), jnp.float32)
    ones16 = jnp.ones((16,), jnp.float32)

    @plsc.parallel_loop(0, HR, unroll=4)
    def zbody(j):
        for u in range(8):
            hcnt[j, pl.ds(u * 16, 16)] = zeros16
            hsum[j, pl.ds(u * 16, 16)] = zeros16

    bufs = (buf0, buf1)
    sems = (sem0, sem1)

    def start(ci):
        return pltpu.async_copy(
            nl_hbm.at[pl.ds(base + ci * CHUNK, CHUNK)],
            bufs[ci % 2], sems[ci % 2])

    def process(buf):
        @plsc.parallel_loop(0, CHUNK // 32, unroll=UNROLL)
        def ibody(i):
            v32 = buf[pl.ds(i * 32, 32)]
            for v in plsc.unpack(v32, format=plsc.PackFormat.INTERLEAVED):
                m = v > 0.0
                bits = plsc.bitcast(v, jnp.int32)
                idx = lax.shift_right_logical(bits, SHIFT)
                hi = lax.shift_right_logical(idx, 7)
                lo = idx & 127
                plsc.addupdate_scatter(hcnt, [hi, lo], ones16, mask=m)
                plsc.addupdate_scatter(hsum, [hi, lo], v, mask=m)

    copies = [None, None]
    copies[0] = start(0)
    for ci in range(NCHUNK):
        if ci + 1 < NCHUNK:
            copies[(ci + 1) % 2] = start(ci + 1)
        copies[ci % 2].wait()
        process(bufs[ci % 2])

    pltpu.sync_copy(hcnt, ocnt_hbm.at[wid])
    pltpu.sync_copy(hsum, osum_hbm.at[wid])


def _sc_hist(nl_flat):
    mesh = plsc.VectorSubcoreMesh(core_axis_name="c", subcore_axis_name="s")
    f = pl.kernel(
        _sc_hist_body,
        out_type=[
            jax.ShapeDtypeStruct((NW, HR, 128), jnp.float32),
            jax.ShapeDtypeStruct((NW, HR, 128), jnp.float32),
        ],
        mesh=mesh,
        compiler_params=pltpu.CompilerParams(needs_layout_passes=False),
        scratch_types=[
            pltpu.VMEM((CHUNK,), jnp.bfloat16),
            pltpu.VMEM((CHUNK,), jnp.bfloat16),
            pltpu.VMEM((HR, 128), jnp.float32),
            pltpu.VMEM((HR, 128), jnp.float32),
            pltpu.SemaphoreType.DMA,
            pltpu.SemaphoreType.DMA,
        ],
    )
    return f(nl_flat)


def _k3_select(cnt_ref, sm_ref, sp_ref, np_ref, out_ref):
    npos = np_ref[0, 0]
    cnt = jnp.sum(cnt_ref[...], axis=0)
    sm = jnp.sum(sm_ref[...], axis=0)

    col = lax.broadcasted_iota(jnp.int32, (128, 128), 0)
    row = lax.broadcasted_iota(jnp.int32, (128, 128), 1)
    upper = (col <= row).astype(jnp.float32)
    incl_row = lax.dot(cnt, upper, precision=lax.Precision.HIGHEST,
                       preferred_element_type=jnp.float32)
    row_tot = incl_row[:, 127:128]
    i2 = lax.broadcasted_iota(jnp.int32, (HR, HR), 0)
    j2 = lax.broadcasted_iota(jnp.int32, (HR, HR), 1)
    lstrict = (j2 < i2).astype(jnp.float32)
    pref_rows = lax.dot(lstrict, row_tot, precision=lax.Precision.HIGHEST,
                        preferred_element_type=jnp.float32)
    incl = incl_row + pref_rows
    tot = jnp.sum(cnt)
    above = tot - incl

    npi = npos.astype(jnp.int32)
    maxneg = N - npi
    k = jnp.minimum(jnp.maximum(MIN_NEG, 5 * npi), maxneg)
    kf = k.astype(jnp.float32)

    w = jnp.clip((kf - above) / cnt, 0.0, 1.0)
    w = jnp.where(cnt > 0.0, w, 0.0)
    sum_neg = jnp.sum(sm * w)
    count = npos + kf
    out_ref[0, 0] = (sp_ref[0, 0] + sum_neg) / count


def kernel(input, target):
    nl, sp, npos = pl.pallas_call(
        _k1_bce,
        grid=(GRID1,),
        in_specs=[
            pl.BlockSpec((BLK1,), lambda i: (i,)),
            pl.BlockSpec((BLK1,), lambda i: (i,)),
        ],
        out_specs=[
            pl.BlockSpec((BLK1,), lambda i: (i,)),
            pl.BlockSpec(memory_space=pltpu.SMEM),
            pl.BlockSpec(memory_space=pltpu.SMEM),
        ],
        out_shape=[
            jax.ShapeDtypeStruct((N,), jnp.bfloat16),
            jax.ShapeDtypeStruct((1, 1), jnp.float32),
            jax.ShapeDtypeStruct((1, 1), jnp.float32),
        ],
    )(input, target)

    ocnt, osum = _sc_hist(nl)

    out = pl.pallas_call(
        _k3_select,
        in_specs=[
            pl.BlockSpec((NW, HR, 128), lambda: (0, 0, 0)),
            pl.BlockSpec((NW, HR, 128), lambda: (0, 0, 0)),
            pl.BlockSpec(memory_space=pltpu.SMEM),
            pl.BlockSpec(memory_space=pltpu.SMEM),
        ],
        out_specs=pl.BlockSpec(memory_space=pltpu.SMEM),
        out_shape=jax.ShapeDtypeStruct((1, 1), jnp.float32),
    )(ocnt, osum, sp, npos)

    return out[0, 0]
